# baseline (speedup 1.0000x reference)
import jax
import jax.numpy as jnp
from jax.experimental import pallas as pl
from jax.experimental.pallas import tpu as pltpu

EPS = 1e-08

_B, _C, _H, _W = 32, 64, 256, 256
_N = _B * _H * _W

_NH = 64
_NGJ = _H // _NH


def _stats_body(x_ref, sum_ref, sq_ref):
    xb = x_ref[...]
    sum_ref[...] = jnp.sum(xb, axis=(1, 2)).reshape(1, _C, 1)
    sq_ref[...] = jnp.sum(xb * xb, axis=(1, 2)).reshape(1, _C, 1)


def _norm_body(ps_ref, pq_ref, x_ref, o_ref):
    total = jnp.sum(ps_ref[...], axis=0)
    totsq = jnp.sum(pq_ref[...], axis=0)
    mean = total / _N
    m2 = totsq - total * mean
    var = jnp.maximum(m2 / (_N - 1), EPS)
    rstd = jax.lax.rsqrt(var + EPS)
    o_ref[...] = (x_ref[...] - mean[:, :, None]) * rstd[:, :, None]


def kernel(x):
    x3 = x.reshape(_B * _C, _H, _W)

    ps, pq = pl.pallas_call(
        _stats_body,
        grid=(_B,),
        in_specs=[pl.BlockSpec((_C, _H, _W), lambda i: (i, 0, 0))],
        out_specs=[
            pl.BlockSpec((1, _C, 1), lambda i: (i, 0, 0)),
            pl.BlockSpec((1, _C, 1), lambda i: (i, 0, 0)),
        ],
        out_shape=[
            jax.ShapeDtypeStruct((_B, _C, 1), jnp.float32),
            jax.ShapeDtypeStruct((_B, _C, 1), jnp.float32),
        ],
        compiler_params=pltpu.CompilerParams(
            dimension_semantics=("parallel",),
            vmem_limit_bytes=48 * 1024 * 1024,
        ),
        name="welford_stats",
    )(x3)

    z3 = pl.pallas_call(
        _norm_body,
        grid=(_B, _NGJ),
        in_specs=[
            pl.BlockSpec((_B, _C, 1), lambda i, j: (0, 0, 0)),
            pl.BlockSpec((_B, _C, 1), lambda i, j: (0, 0, 0)),
            pl.BlockSpec((_C, _NH, _W), lambda i, j: (i, j, 0)),
        ],
        out_specs=pl.BlockSpec((_C, _NH, _W), lambda i, j: (i, j, 0)),
        out_shape=jax.ShapeDtypeStruct((_B * _C, _H, _W), jnp.float32),
        compiler_params=pltpu.CompilerParams(
            dimension_semantics=("parallel", "arbitrary"),
            vmem_limit_bytes=48 * 1024 * 1024,
        ),
        name="welford_normalize",
    )(ps, pq, x3)

    return z3.reshape(x.shape)


# speedup vs baseline: 1.0147x; 1.0147x over previous
import jax
import jax.numpy as jnp
from jax.experimental import pallas as pl
from jax.experimental.pallas import tpu as pltpu

EPS = 1e-08

_B, _C, _H, _W = 32, 64, 256, 256
_N = _B * _H * _W

_NH = 128
_GJ = _H // _NH


def _body(x_ref, o_ref, acc_s, acc_q):
    p = pl.program_id(0)
    i = pl.program_id(1)
    j = pl.program_id(2)

    @pl.when((p == 0) & (i == 0) & (j == 0))
    def _init():
        acc_s[...] = jnp.zeros_like(acc_s)
        acc_q[...] = jnp.zeros_like(acc_q)

    @pl.when(p == 0)
    def _stats():
        xb = x_ref[...]
        acc_s[...] += jnp.sum(xb, axis=(1, 2)).reshape(_C, 1)
        acc_q[...] += jnp.sum(xb * xb, axis=(1, 2)).reshape(_C, 1)

    @pl.when(p == 1)
    def _norm():
        total = acc_s[...]
        mean = total / _N
        m2 = acc_q[...] - total * mean
        var = jnp.maximum(m2 / (_N - 1), EPS)
        rstd = jax.lax.rsqrt(var + EPS)
        o_ref[...] = (x_ref[...] - mean[:, :, None]) * rstd[:, :, None]


def kernel(x):
    x3 = x.reshape(_B * _C, _H, _W)

    z3 = pl.pallas_call(
        _body,
        grid=(2, _B, _GJ),
        in_specs=[pl.BlockSpec((_C, _NH, _W), lambda p, i, j: (i, j, 0))],
        out_specs=pl.BlockSpec((_C, _NH, _W),
                               lambda p, i, j: (i * p, j * p, 0)),
        out_shape=jax.ShapeDtypeStruct((_B * _C, _H, _W), jnp.float32),
        scratch_shapes=[
            pltpu.VMEM((_C, 1), jnp.float32),
            pltpu.VMEM((_C, 1), jnp.float32),
        ],
        compiler_params=pltpu.CompilerParams(
            dimension_semantics=("arbitrary", "arbitrary", "arbitrary"),
            vmem_limit_bytes=48 * 1024 * 1024,
        ),
        name="welford_fused",
    )(x3)

    return z3.reshape(x.shape)
